# Initial kernel scaffold; baseline (speedup 1.0000x reference)
import jax
import jax.numpy as jnp
from jax.experimental import pallas as pl


def kernel(x, edge_index, edge_attr, W_src, W_dst, W_edge, t, W_mlp1, bn_gamma, bn_beta, W_mlp2, ln_gamma, ln_beta):
    raise NotImplementedError("write your pallas kernel here")


# speedup vs baseline: 1.7191x; 1.7191x over previous
import functools

import jax
import jax.numpy as jnp
import numpy as np
from jax import lax
from jax.experimental import pallas as pl
from jax.experimental.pallas import tpu as pltpu
from jax.experimental.pallas import tpu_sc as plsc

N = 10000
E = 320000
D_IN = 128
D_HID = 256
D_EDGE = 16

NC = 2
NS = 16
NW = NC * NS
EPW = E // NW
BATCH = 80
NB = EPW // BATCH
NCHUNK = 4
CCH = D_HID // NCHUNK
RPT = N // NS

_F32 = jnp.float32


def _sds(shape, dtype=_F32):
    return jax.ShapeDtypeStruct(shape, dtype)


def _tc0_body(x_ref, ws_ref, wd_ref, xs0, xs1, xs2, xs3, xd_ref):
    x = x_ref[...]
    xs = jnp.dot(x, ws_ref[...], preferred_element_type=_F32)
    outs = (xs0, xs1, xs2, xs3)
    for k in range(NCHUNK):
        outs[k][...] = xs[:, k * CCH:(k + 1) * CCH]
    xd_ref[...] = jnp.dot(x, wd_ref[...], preferred_element_type=_F32)


def _tc0(x, w_src, w_dst):
    blk = 400
    grid = N // blk
    return pl.pallas_call(
        _tc0_body,
        grid=(grid,),
        in_specs=[
            pl.BlockSpec((blk, D_IN), lambda i: (i, 0)),
            pl.BlockSpec((D_IN, D_HID), lambda i: (0, 0)),
            pl.BlockSpec((D_IN, D_HID), lambda i: (0, 0)),
        ],
        out_specs=[pl.BlockSpec((blk, CCH), lambda i: (i, 0))] * NCHUNK
        + [pl.BlockSpec((blk, D_HID), lambda i: (i, 0))],
        out_shape=[_sds((N, CCH))] * NCHUNK + [_sds((N, D_HID))],
    )(x, w_src, w_dst)


def _tc1_body(ea_ref, we_ref, e0, e1, e2, e3):
    e = jnp.dot(ea_ref[...], we_ref[...], preferred_element_type=_F32)
    outs = (e0, e1, e2, e3)
    for k in range(NCHUNK):
        outs[k][...] = e[:, k * CCH:(k + 1) * CCH]


def _tc1(edge_attr, w_edge):
    blk = 8000
    grid = E // blk
    return pl.pallas_call(
        _tc1_body,
        grid=(grid,),
        in_specs=[
            pl.BlockSpec((blk, D_EDGE), lambda i: (i, 0)),
            pl.BlockSpec((D_EDGE, D_HID), lambda i: (0, 0)),
        ],
        out_specs=[pl.BlockSpec((blk, CCH), lambda i: (i, 0))] * NCHUNK,
        out_shape=[_sds((E, CCH))] * NCHUNK,
    )(edge_attr, w_edge)


def _edge_stage_jnp(xs, es, src, dst, t):
    x_src = jnp.concatenate(xs, axis=1)
    e = jnp.concatenate(es, axis=1)
    msg = jnp.maximum(x_src[src] + e, 0.0) + 1e-7
    p = jnp.exp(msg * t)
    num = jax.ops.segment_sum(p * msg, dst, num_segments=N)
    den = jax.ops.segment_sum(p, dst, num_segments=N)
    return num[None], den[None]


def _tc2_body(num_ref, den_ref, xd_ref, w1_ref, bg_ref, bb_ref, w2_ref,
              h_ref, s_ref, acc):
    i = pl.program_id(0)
    n = jnp.sum(num_ref[...], axis=0)
    d = jnp.sum(den_ref[...], axis=0) + 1e-16
    outb = n / d + xd_ref[...]
    h1 = jnp.dot(outb, w1_ref[...], preferred_element_type=_F32)
    bn_scale = bg_ref[...] * np.float32(1.0 / np.sqrt(1.0 + 1e-5))
    h1 = jnp.maximum(h1 * bn_scale + bb_ref[...], 0.0)
    h2 = jnp.dot(h1, w2_ref[...], preferred_element_type=_F32)
    h2 = jnp.maximum(h2, 0.0)
    h_ref[...] = h2
    s1 = jnp.sum(h2)
    s2 = jnp.sum(h2 * h2)

    @pl.when(i == 0)
    def _():
        acc[0] = s1
        acc[1] = s2

    @pl.when(i != 0)
    def _():
        acc[0] += s1
        acc[1] += s2

    @pl.when(i == pl.num_programs(0) - 1)
    def _():
        lane = lax.broadcasted_iota(jnp.int32, (1, 128), 1)
        s_ref[...] = jnp.where(lane == 0, acc[0],
                               jnp.where(lane == 1, acc[1], 0.0))


def _tc2(num, den, x_dst, w1, bn_gamma, bn_beta, w2):
    blk = 400
    grid = N // blk
    ncores = num.shape[0]
    return pl.pallas_call(
        _tc2_body,
        grid=(grid,),
        in_specs=[
            pl.BlockSpec((ncores, blk, D_HID), lambda i: (0, i, 0)),
            pl.BlockSpec((ncores, blk, D_HID), lambda i: (0, i, 0)),
            pl.BlockSpec((blk, D_HID), lambda i: (i, 0)),
            pl.BlockSpec((D_HID, 2 * D_HID), lambda i: (0, 0)),
            pl.BlockSpec((1, 2 * D_HID), lambda i: (0, 0)),
            pl.BlockSpec((1, 2 * D_HID), lambda i: (0, 0)),
            pl.BlockSpec((2 * D_HID, D_HID), lambda i: (0, 0)),
        ],
        out_specs=[
            pl.BlockSpec((blk, D_HID), lambda i: (i, 0)),
            pl.BlockSpec((1, 128), lambda i: (0, 0)),
        ],
        out_shape=[_sds((N, D_HID)), _sds((1, 128))],
        scratch_shapes=[pltpu.SMEM((2,), _F32)],
    )(num, den, x_dst, w1, bn_gamma.reshape(1, -1), bn_beta.reshape(1, -1), w2)


def _tc3_body(s_ref, h_ref, g_ref, b_ref, o_ref):
    cnt = np.float32(N * D_HID)
    mean = s_ref[0, 0] / cnt
    var = s_ref[0, 1] / cnt - mean * mean
    std = jnp.sqrt(jnp.maximum(var, 0.0))
    inv = 1.0 / (std + 1e-5)
    o_ref[...] = (h_ref[...] - mean) * inv * g_ref[...] + b_ref[...]


def _tc3(s, h, ln_gamma, ln_beta):
    blk = 1000
    grid = N // blk
    return pl.pallas_call(
        _tc3_body,
        grid=(grid,),
        in_specs=[
            pl.BlockSpec(memory_space=pltpu.SMEM),
            pl.BlockSpec((blk, D_HID), lambda i: (i, 0)),
            pl.BlockSpec((1, D_HID), lambda i: (0, 0)),
            pl.BlockSpec((1, D_HID), lambda i: (0, 0)),
        ],
        out_specs=pl.BlockSpec((blk, D_HID), lambda i: (i, 0)),
        out_shape=_sds((N, D_HID)),
    )(s, h, ln_gamma.reshape(1, -1), ln_beta.reshape(1, -1))


def kernel(x, edge_index, edge_attr, W_src, W_dst, W_edge, t,
           W_mlp1, bn_gamma, bn_beta, W_mlp2, ln_gamma, ln_beta):
    src = edge_index[0]
    dst = edge_index[1]
    xs0, xs1, xs2, xs3, x_dst = _tc0(x, W_src, W_dst)
    e0, e1, e2, e3 = _tc1(edge_attr, W_edge)
    num, den = _edge_stage_jnp((xs0, xs1, xs2, xs3), (e0, e1, e2, e3),
                               src, dst, t)
    h_pre, s = _tc2(num, den, x_dst, W_mlp1, bn_gamma, bn_beta, W_mlp2)
    return _tc3(s, h_pre, ln_gamma, ln_beta)


# speedup vs baseline: 2.0904x; 1.2159x over previous
import functools

import jax
import jax.numpy as jnp
import numpy as np
from jax import lax
from jax.experimental import pallas as pl
from jax.experimental.pallas import tpu as pltpu
from jax.experimental.pallas import tpu_sc as plsc

N = 10000
E = 320000
D_IN = 128
D_HID = 256
D_EDGE = 16

NC = 2
NS = 16
NW = NC * NS
EPW = E // NW
BATCH = 80
NB = EPW // BATCH
NCHUNK = 4
CCH = D_HID // NCHUNK
NPC = 5120
DUMMY = NPC
ACC_ROWS = NPC + 128
RPT = NPC // NS
EPT = E // NS
NBT = EPT // BATCH

_F32 = jnp.float32


def _sds(shape, dtype=_F32):
    return jax.ShapeDtypeStruct(shape, dtype)


def _tc0_body(x_ref, ws_ref, wd_ref, xsa, xsb, xd_ref):
    x = x_ref[...]
    xs = jnp.dot(x, ws_ref[...], preferred_element_type=_F32)
    xsa[...] = xs[:, :128]
    xsb[...] = xs[:, 128:]
    xd_ref[...] = jnp.dot(x, wd_ref[...], preferred_element_type=_F32)


def _tc0(x, w_src, w_dst):
    blk = 400
    grid = N // blk
    return pl.pallas_call(
        _tc0_body,
        grid=(grid,),
        in_specs=[
            pl.BlockSpec((blk, D_IN), lambda i: (i, 0)),
            pl.BlockSpec((D_IN, D_HID), lambda i: (0, 0)),
            pl.BlockSpec((D_IN, D_HID), lambda i: (0, 0)),
        ],
        out_specs=[pl.BlockSpec((blk, 128), lambda i: (i, 0))] * 2
        + [pl.BlockSpec((blk, D_HID), lambda i: (i, 0))],
        out_shape=[_sds((N, 128))] * 2 + [_sds((N, D_HID))],
    )(x, w_src, w_dst)


def _tc1_body(ea_ref, we_ref, ea_out, eb_out):
    e = jnp.dot(ea_ref[...], we_ref[...], preferred_element_type=_F32)
    ea_out[...] = e[:, :128]
    eb_out[...] = e[:, 128:]


def _tc1(edge_attr, w_edge):
    blk = 8000
    grid = E // blk
    return pl.pallas_call(
        _tc1_body,
        grid=(grid,),
        in_specs=[
            pl.BlockSpec((blk, D_EDGE), lambda i: (i, 0)),
            pl.BlockSpec((D_EDGE, D_HID), lambda i: (0, 0)),
        ],
        out_specs=[pl.BlockSpec((blk, 128), lambda i: (i, 0))] * 2,
        out_shape=[_sds((E, 128))] * 2,
    )(edge_attr, w_edge)


def _sc_edge(xs, es, src, dst, t16):
    mesh = plsc.VectorSubcoreMesh(core_axis_name="c", subcore_axis_name="s",
                                  num_cores=NC, num_subcores=NS)
    zeros = jnp.zeros((160, 128), _F32)

    @functools.partial(
        pl.kernel,
        out_type=_sds((NC, NCHUNK, NPC, 128)),
        mesh=mesh,
        scratch_types=[
            pltpu.VMEM((BATCH,), jnp.int32),
            pltpu.VMEM((BATCH,), jnp.int32),
            pltpu.VMEM((BATCH,), jnp.int32),
            pltpu.VMEM((BATCH, 128), _F32),
            pltpu.VMEM((BATCH, 128), _F32),
            pltpu.VMEM((BATCH, 128), _F32),
            pltpu.VMEM((16,), _F32),
            pltpu.VMEM((160, 128), _F32),
            pltpu.VMEM_SHARED((ACC_ROWS, 128), _F32),
            pltpu.SemaphoreType.DMA,
        ],
    )
    def sck(xsa, xsb, ea, eb, src_h, dst_h, t_h, z_h,
            po, sidx, didx, didx2, gbuf, ebuf, stage, tv, zbuf,
            acc, sem):
        c = lax.axis_index("c")
        s = lax.axis_index("s")
        base = s * EPT
        nbase = c * NPC
        pltpu.sync_copy(t_h, tv)
        pltpu.sync_copy(z_h, zbuf)
        tvec = tv[...]
        rbase = s * RPT
        xs_l = (xsa, xsa, xsb, xsb)
        e_l = (ea, ea, eb, eb)
        for k in range(NCHUNK):
            for z in range(2):
                pltpu.sync_copy(zbuf, acc.at[pl.ds(rbase + z * 160, 160)])
            plsc.subcore_barrier()

            def batch_body(j, carry, k=k):
                off = j * BATCH
                pltpu.sync_copy(src_h.at[pl.ds(base + off, BATCH)], sidx)
                pltpu.sync_copy(dst_h.at[pl.ds(base + off, BATCH)], didx)
                gcp = pltpu.async_copy(xs_l[k].at[sidx], gbuf, sem)
                pltpu.sync_copy(e_l[k].at[pl.ds(base + off, BATCH)], ebuf)
                for q in range(BATCH // 16):
                    dv = didx[pl.ds(q * 16, 16)] - nbase
                    ok = (dv >= 0) & (dv < NPC)
                    didx2[pl.ds(q * 16, 16)] = jnp.where(ok, dv, DUMMY)
                gcp.wait()
                loff = (k % 2) * CCH

                def edge_body(i, cc):
                    for v in range(CCH // 16):
                        g = gbuf[i, pl.ds(loff + v * 16, 16)]
                        e = ebuf[i, pl.ds(loff + v * 16, 16)]
                        m = jnp.maximum(g + e, 0.0) + 1e-7
                        p = jnp.exp(m * tvec)
                        stage[i, pl.ds(v * 16, 16)] = p * m
                        stage[i, pl.ds(CCH + v * 16, 16)] = p
                    return cc

                lax.fori_loop(0, BATCH, edge_body, 0)
                pltpu.sync_copy(stage, acc.at[didx2], add=True)
                return carry

            lax.fori_loop(0, NBT, batch_body, 0)
            plsc.subcore_barrier()
            pltpu.sync_copy(acc.at[pl.ds(rbase, RPT)],
                            po.at[c, k, pl.ds(rbase, RPT)])
            plsc.subcore_barrier()

    return sck(xs[0], xs[1], es[0], es[1], src, dst, t16, zeros)


def _tc2_body(po_ref, xd_ref, w1_ref, bg_ref, bb_ref, w2_ref,
              h_ref, s_ref, acc):
    i = pl.program_id(0)
    po = po_ref[...]
    n0 = jnp.concatenate([po[0, k, :, :CCH] for k in range(NCHUNK)], axis=-1)
    d0 = jnp.concatenate([po[0, k, :, CCH:] for k in range(NCHUNK)], axis=-1)
    n1 = jnp.concatenate([po[1, k, :, :CCH] for k in range(NCHUNK)], axis=-1)
    d1 = jnp.concatenate([po[1, k, :, CCH:] for k in range(NCHUNK)], axis=-1)
    c1 = i >= (NPC // 80)
    n = jnp.where(c1, n1, n0)
    d = jnp.where(c1, d1, d0) + 1e-16
    outb = n / d + xd_ref[...]
    h1 = jnp.dot(outb, w1_ref[...], preferred_element_type=_F32)
    bn_scale = bg_ref[...] * np.float32(1.0 / np.sqrt(1.0 + 1e-5))
    h1 = jnp.maximum(h1 * bn_scale + bb_ref[...], 0.0)
    h2 = jnp.dot(h1, w2_ref[...], preferred_element_type=_F32)
    h2 = jnp.maximum(h2, 0.0)
    h_ref[...] = h2
    s1 = jnp.sum(h2)
    s2 = jnp.sum(h2 * h2)

    @pl.when(i == 0)
    def _():
        acc[0] = s1
        acc[1] = s2

    @pl.when(i != 0)
    def _():
        acc[0] += s1
        acc[1] += s2

    @pl.when(i == pl.num_programs(0) - 1)
    def _():
        lane = lax.broadcasted_iota(jnp.int32, (1, 128), 1)
        s_ref[...] = jnp.where(lane == 0, acc[0],
                               jnp.where(lane == 1, acc[1], 0.0))


def _tc2(po, x_dst, w1, bn_gamma, bn_beta, w2):
    blk = 80
    grid = N // blk
    nblk = NPC // blk

    def po_map(i):
        return (0, 0, jnp.where(i < nblk, i, i - nblk), 0)

    return pl.pallas_call(
        _tc2_body,
        grid=(grid,),
        in_specs=[
            pl.BlockSpec((NC, NCHUNK, blk, 128), po_map),
            pl.BlockSpec((blk, D_HID), lambda i: (i, 0)),
            pl.BlockSpec((D_HID, 2 * D_HID), lambda i: (0, 0)),
            pl.BlockSpec((1, 2 * D_HID), lambda i: (0, 0)),
            pl.BlockSpec((1, 2 * D_HID), lambda i: (0, 0)),
            pl.BlockSpec((2 * D_HID, D_HID), lambda i: (0, 0)),
        ],
        out_specs=[
            pl.BlockSpec((blk, D_HID), lambda i: (i, 0)),
            pl.BlockSpec((1, 128), lambda i: (0, 0)),
        ],
        out_shape=[_sds((N, D_HID)), _sds((1, 128))],
        scratch_shapes=[pltpu.SMEM((2,), _F32)],
    )(po, x_dst, w1, bn_gamma.reshape(1, -1), bn_beta.reshape(1, -1), w2)


def _tc3_body(s_ref, h_ref, g_ref, b_ref, o_ref):
    cnt = np.float32(N * D_HID)
    mean = s_ref[0, 0] / cnt
    var = s_ref[0, 1] / cnt - mean * mean
    std = jnp.sqrt(jnp.maximum(var, 0.0))
    inv = 1.0 / (std + 1e-5)
    o_ref[...] = (h_ref[...] - mean) * inv * g_ref[...] + b_ref[...]


def _tc3(s, h, ln_gamma, ln_beta):
    blk = 1000
    grid = N // blk
    return pl.pallas_call(
        _tc3_body,
        grid=(grid,),
        in_specs=[
            pl.BlockSpec(memory_space=pltpu.SMEM),
            pl.BlockSpec((blk, D_HID), lambda i: (i, 0)),
            pl.BlockSpec((1, D_HID), lambda i: (0, 0)),
            pl.BlockSpec((1, D_HID), lambda i: (0, 0)),
        ],
        out_specs=pl.BlockSpec((blk, D_HID), lambda i: (i, 0)),
        out_shape=_sds((N, D_HID)),
    )(s, h, ln_gamma.reshape(1, -1), ln_beta.reshape(1, -1))


def kernel(x, edge_index, edge_attr, W_src, W_dst, W_edge, t,
           W_mlp1, bn_gamma, bn_beta, W_mlp2, ln_gamma, ln_beta):
    src = edge_index[0]
    dst = edge_index[1]
    t16 = jnp.full((16,), t, _F32)
    xsa, xsb, x_dst = _tc0(x, W_src, W_dst)
    ea, eb = _tc1(edge_attr, W_edge)
    po = _sc_edge((xsa, xsb), (ea, eb), src, dst, t16)
    h_pre, s = _tc2(po, x_dst, W_mlp1, bn_gamma, bn_beta, W_mlp2)
    return _tc3(s, h_pre, ln_gamma, ln_beta)


# speedup vs baseline: 2.4079x; 1.1519x over previous
import functools

import jax
import jax.numpy as jnp
import numpy as np
from jax import lax
from jax.experimental import pallas as pl
from jax.experimental.pallas import tpu as pltpu
from jax.experimental.pallas import tpu_sc as plsc

N = 10000
E = 320000
D_IN = 128
D_HID = 256
D_EDGE = 16

NC = 2
NS = 16
NW = NC * NS
EPW = E // NW
BATCH = 128
NB = EPW // BATCH
NCHUNK = 4
CCH = D_HID // NCHUNK
NPC = 5120
DUMMY = NPC
ACC_ROWS = NPC + 128
RPT = NPC // NS
EPT = E // NS
NBT = -(-EPT // BATCH)

_F32 = jnp.float32


def _sds(shape, dtype=_F32):
    return jax.ShapeDtypeStruct(shape, dtype)


def _tc0_body(x_ref, ws_ref, wd_ref, xsa, xsb, xd_ref):
    x = x_ref[...]
    xs = jnp.dot(x, ws_ref[...], preferred_element_type=_F32)
    xsa[...] = xs[:, :128]
    xsb[...] = xs[:, 128:]
    xd_ref[...] = jnp.dot(x, wd_ref[...], preferred_element_type=_F32)


def _tc0(x, w_src, w_dst):
    blk = 400
    grid = N // blk
    return pl.pallas_call(
        _tc0_body,
        grid=(grid,),
        in_specs=[
            pl.BlockSpec((blk, D_IN), lambda i: (i, 0)),
            pl.BlockSpec((D_IN, D_HID), lambda i: (0, 0)),
            pl.BlockSpec((D_IN, D_HID), lambda i: (0, 0)),
        ],
        out_specs=[pl.BlockSpec((blk, 128), lambda i: (i, 0))] * 2
        + [pl.BlockSpec((blk, D_HID), lambda i: (i, 0))],
        out_shape=[_sds((N, 128))] * 2 + [_sds((N, D_HID))],
    )(x, w_src, w_dst)


def _tc1_body(ea_ref, we_ref, ea_out, eb_out):
    e = jnp.dot(ea_ref[...], we_ref[...], preferred_element_type=_F32)
    ea_out[...] = e[:, :128]
    eb_out[...] = e[:, 128:]


def _tc1(edge_attr, w_edge):
    blk = 8000
    grid = E // blk
    return pl.pallas_call(
        _tc1_body,
        grid=(grid,),
        in_specs=[
            pl.BlockSpec((blk, D_EDGE), lambda i: (i, 0)),
            pl.BlockSpec((D_EDGE, D_HID), lambda i: (0, 0)),
        ],
        out_specs=[pl.BlockSpec((blk, 128), lambda i: (i, 0))] * 2,
        out_shape=[_sds((E + 128, 128))] * 2,
    )(edge_attr, w_edge)


def _sc_edge(xs, es, src, dst, t16):
    mesh = plsc.VectorSubcoreMesh(core_axis_name="c", subcore_axis_name="s",
                                  num_cores=NC, num_subcores=NS)
    zeros = jnp.zeros((160, 128), _F32)

    @functools.partial(
        pl.kernel,
        out_type=_sds((NC, NCHUNK, NPC, 128)),
        mesh=mesh,
        scratch_types=[
            pltpu.VMEM((BATCH,), jnp.int32),
            pltpu.VMEM((BATCH,), jnp.int32),
            pltpu.VMEM((BATCH,), jnp.int32),
            pltpu.VMEM((BATCH, 128), _F32),
            pltpu.VMEM((BATCH, 128), _F32),
            pltpu.VMEM((BATCH, 128), _F32),
            pltpu.VMEM((16,), _F32),
            pltpu.VMEM((160, 128), _F32),
            pltpu.VMEM_SHARED((ACC_ROWS, 128), _F32),
            pltpu.SemaphoreType.DMA,
        ],
    )
    def sck(xsa, xsb, ea, eb, src_h, dst_h, t_h, z_h,
            po, sidx, didx, didx2, gbuf, ebuf, stage, tv, zbuf,
            acc, sem):
        c = lax.axis_index("c")
        s = lax.axis_index("s")
        base = s * EPT
        nbase = c * NPC
        pltpu.sync_copy(t_h, tv)
        pltpu.sync_copy(z_h, zbuf)
        tvec = tv[...]
        rbase = s * RPT
        xs_l = (xsa, xsa, xsb, xsb)
        e_l = (ea, ea, eb, eb)
        for k in range(NCHUNK):
            for z in range(2):
                pltpu.sync_copy(zbuf, acc.at[pl.ds(rbase + z * 160, 160)])
            plsc.subcore_barrier()

            def batch_body(j, carry, k=k):
                off = j * BATCH
                pltpu.sync_copy(src_h.at[pl.ds(base + off, BATCH)], sidx)
                pltpu.sync_copy(dst_h.at[pl.ds(base + off, BATCH)], didx)
                gcp = pltpu.async_copy(xs_l[k].at[sidx], gbuf, sem)
                pltpu.sync_copy(e_l[k].at[pl.ds(base + off, BATCH)], ebuf)
                for q in range(BATCH // 16):
                    dv = didx[pl.ds(q * 16, 16)] - nbase
                    pos = lax.iota(jnp.int32, 16) + (off + q * 16)
                    ok = (dv >= 0) & (dv < NPC) & (pos < EPT)
                    didx2[pl.ds(q * 16, 16)] = jnp.where(ok, dv, DUMMY)
                gcp.wait()
                loff = (k % 2) * CCH

                def edge_body(i, cc):
                    for v in range(CCH // 16):
                        g = gbuf[i, pl.ds(loff + v * 16, 16)]
                        e = ebuf[i, pl.ds(loff + v * 16, 16)]
                        m = jnp.maximum(g + e, 0.0) + 1e-7
                        p = jnp.exp(m * tvec)
                        stage[i, pl.ds(v * 16, 16)] = p * m
                        stage[i, pl.ds(CCH + v * 16, 16)] = p
                    return cc

                lax.fori_loop(0, BATCH, edge_body, 0)
                pltpu.sync_copy(stage, acc.at[didx2], add=True)
                return carry

            lax.fori_loop(0, NBT, batch_body, 0)
            plsc.subcore_barrier()
            pltpu.sync_copy(acc.at[pl.ds(rbase, RPT)],
                            po.at[c, k, pl.ds(rbase, RPT)])
            plsc.subcore_barrier()

    return sck(xs[0], xs[1], es[0], es[1], src, dst, t16, zeros)


def _tc2_body(po_ref, xd_ref, w1_ref, bg_ref, bb_ref, w2_ref,
              h_ref, s_ref, acc):
    i = pl.program_id(0)
    po = po_ref[...]
    n0 = jnp.concatenate([po[0, k, :, :CCH] for k in range(NCHUNK)], axis=-1)
    d0 = jnp.concatenate([po[0, k, :, CCH:] for k in range(NCHUNK)], axis=-1)
    n1 = jnp.concatenate([po[1, k, :, :CCH] for k in range(NCHUNK)], axis=-1)
    d1 = jnp.concatenate([po[1, k, :, CCH:] for k in range(NCHUNK)], axis=-1)
    c1 = i >= (NPC // 80)
    n = jnp.where(c1, n1, n0)
    d = jnp.where(c1, d1, d0) + 1e-16
    outb = n / d + xd_ref[...]
    h1 = jnp.dot(outb, w1_ref[...], preferred_element_type=_F32)
    bn_scale = bg_ref[...] * np.float32(1.0 / np.sqrt(1.0 + 1e-5))
    h1 = jnp.maximum(h1 * bn_scale + bb_ref[...], 0.0)
    h2 = jnp.dot(h1, w2_ref[...], preferred_element_type=_F32)
    h2 = jnp.maximum(h2, 0.0)
    h_ref[...] = h2
    s1 = jnp.sum(h2)
    s2 = jnp.sum(h2 * h2)

    @pl.when(i == 0)
    def _():
        acc[0] = s1
        acc[1] = s2

    @pl.when(i != 0)
    def _():
        acc[0] += s1
        acc[1] += s2

    @pl.when(i == pl.num_programs(0) - 1)
    def _():
        lane = lax.broadcasted_iota(jnp.int32, (1, 128), 1)
        s_ref[...] = jnp.where(lane == 0, acc[0],
                               jnp.where(lane == 1, acc[1], 0.0))


def _tc2(po, x_dst, w1, bn_gamma, bn_beta, w2):
    blk = 80
    grid = N // blk
    nblk = NPC // blk

    def po_map(i):
        return (0, 0, jnp.where(i < nblk, i, i - nblk), 0)

    return pl.pallas_call(
        _tc2_body,
        grid=(grid,),
        in_specs=[
            pl.BlockSpec((NC, NCHUNK, blk, 128), po_map),
            pl.BlockSpec((blk, D_HID), lambda i: (i, 0)),
            pl.BlockSpec((D_HID, 2 * D_HID), lambda i: (0, 0)),
            pl.BlockSpec((1, 2 * D_HID), lambda i: (0, 0)),
            pl.BlockSpec((1, 2 * D_HID), lambda i: (0, 0)),
            pl.BlockSpec((2 * D_HID, D_HID), lambda i: (0, 0)),
        ],
        out_specs=[
            pl.BlockSpec((blk, D_HID), lambda i: (i, 0)),
            pl.BlockSpec((1, 128), lambda i: (0, 0)),
        ],
        out_shape=[_sds((N, D_HID)), _sds((1, 128))],
        scratch_shapes=[pltpu.SMEM((2,), _F32)],
    )(po, x_dst, w1, bn_gamma.reshape(1, -1), bn_beta.reshape(1, -1), w2)


def _tc3_body(s_ref, h_ref, g_ref, b_ref, o_ref):
    cnt = np.float32(N * D_HID)
    mean = s_ref[0, 0] / cnt
    var = s_ref[0, 1] / cnt - mean * mean
    std = jnp.sqrt(jnp.maximum(var, 0.0))
    inv = 1.0 / (std + 1e-5)
    o_ref[...] = (h_ref[...] - mean) * inv * g_ref[...] + b_ref[...]


def _tc3(s, h, ln_gamma, ln_beta):
    blk = 1000
    grid = N // blk
    return pl.pallas_call(
        _tc3_body,
        grid=(grid,),
        in_specs=[
            pl.BlockSpec(memory_space=pltpu.SMEM),
            pl.BlockSpec((blk, D_HID), lambda i: (i, 0)),
            pl.BlockSpec((1, D_HID), lambda i: (0, 0)),
            pl.BlockSpec((1, D_HID), lambda i: (0, 0)),
        ],
        out_specs=pl.BlockSpec((blk, D_HID), lambda i: (i, 0)),
        out_shape=_sds((N, D_HID)),
    )(s, h, ln_gamma.reshape(1, -1), ln_beta.reshape(1, -1))


def kernel(x, edge_index, edge_attr, W_src, W_dst, W_edge, t,
           W_mlp1, bn_gamma, bn_beta, W_mlp2, ln_gamma, ln_beta):
    src = jnp.pad(edge_index[0], (0, 128))
    dst = jnp.pad(edge_index[1], (0, 128), constant_values=N)
    t16 = jnp.full((16,), t, _F32)
    xsa, xsb, x_dst = _tc0(x, W_src, W_dst)
    ea, eb = _tc1(edge_attr, W_edge)
    po = _sc_edge((xsa, xsb), (ea, eb), src, dst, t16)
    h_pre, s = _tc2(po, x_dst, W_mlp1, bn_gamma, bn_beta, W_mlp2)
    return _tc3(s, h_pre, ln_gamma, ln_beta)


# speedup vs baseline: 3.7854x; 1.5721x over previous
import functools

import jax
import jax.numpy as jnp
import numpy as np
from jax import lax
from jax.experimental import pallas as pl
from jax.experimental.pallas import tpu as pltpu
from jax.experimental.pallas import tpu_sc as plsc

N = 10000
E = 320000
D_IN = 128
D_HID = 256
D_EDGE = 16

NC = 2
NS = 16
NW = NC * NS
EPW = E // NW
BATCH = 128
NB = EPW // BATCH
NCHUNK = 4
CCH = D_HID // NCHUNK
NPC = 5120
DUMMY = NPC
ACC_ROWS = NPC + 128
RPT = NPC // NS
EPT = E // NS
NBT = -(-EPT // BATCH)

_F32 = jnp.float32


def _sds(shape, dtype=_F32):
    return jax.ShapeDtypeStruct(shape, dtype)


def _tc0_body(x_ref, ws_ref, wd_ref, xsa, xsb, xd_ref):
    x = x_ref[...]
    xs = jnp.dot(x, ws_ref[...], preferred_element_type=_F32)
    xsa[...] = xs[:, :128]
    xsb[...] = xs[:, 128:]
    xd_ref[...] = jnp.dot(x, wd_ref[...], preferred_element_type=_F32)


def _tc0(x, w_src, w_dst):
    blk = 400
    grid = N // blk
    return pl.pallas_call(
        _tc0_body,
        grid=(grid,),
        in_specs=[
            pl.BlockSpec((blk, D_IN), lambda i: (i, 0)),
            pl.BlockSpec((D_IN, D_HID), lambda i: (0, 0)),
            pl.BlockSpec((D_IN, D_HID), lambda i: (0, 0)),
        ],
        out_specs=[pl.BlockSpec((blk, 128), lambda i: (i, 0))] * 2
        + [pl.BlockSpec((blk, D_HID), lambda i: (i, 0))],
        out_shape=[_sds((N, 128))] * 2 + [_sds((N, D_HID))],
    )(x, w_src, w_dst)


def _tc1_body(ea_ref, we_ref, ea_out, eb_out):
    e = jnp.dot(ea_ref[...], we_ref[...], preferred_element_type=_F32)
    ea_out[...] = e[:, :128]
    eb_out[...] = e[:, 128:]


def _tc1(edge_attr, w_edge):
    blk = 8000
    grid = E // blk
    return pl.pallas_call(
        _tc1_body,
        grid=(grid,),
        in_specs=[
            pl.BlockSpec((blk, D_EDGE), lambda i: (i, 0)),
            pl.BlockSpec((D_EDGE, D_HID), lambda i: (0, 0)),
        ],
        out_specs=[pl.BlockSpec((blk, 128), lambda i: (i, 0))] * 2,
        out_shape=[_sds((E + 128, 128))] * 2,
    )(edge_attr, w_edge)


def _sc_edge(xs, es, src, dst, t16):
    mesh = plsc.VectorSubcoreMesh(core_axis_name="c", subcore_axis_name="s",
                                  num_cores=NC, num_subcores=NS)
    zeros = jnp.zeros((160, 128), _F32)

    @functools.partial(
        pl.kernel,
        out_type=_sds((NC, NCHUNK, NPC, 128)),
        mesh=mesh,
        scratch_types=[
            pltpu.VMEM((BATCH,), jnp.int32),
            pltpu.VMEM((BATCH,), jnp.int32),
            pltpu.VMEM((BATCH,), jnp.int32),
            pltpu.VMEM((BATCH,), jnp.int32),
            pltpu.VMEM((BATCH,), jnp.int32),
            pltpu.VMEM((BATCH, 128), _F32),
            pltpu.VMEM((BATCH, 128), _F32),
            pltpu.VMEM((BATCH, 128), _F32),
            pltpu.VMEM((BATCH, 128), _F32),
            pltpu.VMEM((BATCH, 128), _F32),
            pltpu.VMEM((16,), _F32),
            pltpu.VMEM_SHARED((ACC_ROWS, 128), _F32),
            pltpu.SemaphoreType.DMA,
            pltpu.SemaphoreType.DMA,
            pltpu.SemaphoreType.DMA,
            pltpu.SemaphoreType.DMA,
            pltpu.SemaphoreType.DMA,
            pltpu.SemaphoreType.DMA,
        ],
    )
    def sck(xsa, xsb, ea, eb, src_h, dst_h, t_h, z_h,
            po, sidx0, sidx1, didx0, didx1, didx2, gbuf0, gbuf1,
            ebuf0, ebuf1, stage, tv, acc,
            isem0, isem1, gsem0, gsem1, esem0, esem1):
        c = lax.axis_index("c")
        s = lax.axis_index("s")
        base = s * EPT
        nbase = c * NPC
        pltpu.sync_copy(t_h, tv)
        tvec = tv[...]
        rbase = s * RPT
        xs_l = (xsa, xsa, xsb, xsb)
        e_l = (ea, ea, eb, eb)
        for k in range(NCHUNK):
            for z in range(2):
                pltpu.sync_copy(z_h, acc.at[pl.ds(rbase + z * 160, 160)])
            plsc.subcore_barrier()

            sidx = (sidx0, sidx1)
            didx = (didx0, didx1)
            gbuf = (gbuf0, gbuf1)
            ebuf = (ebuf0, ebuf1)
            isem = (isem0, isem1)
            gsem = (gsem0, gsem1)
            esem = (esem0, esem1)
            loff = (k % 2) * CCH
            tbl = xs_l[k]
            etb = e_l[k]

            def idx_start(j, h):
                off = base + j * BATCH
                c1 = pltpu.async_copy(src_h.at[pl.ds(off, BATCH)],
                                      sidx[h], isem[h])
                c2 = pltpu.async_copy(dst_h.at[pl.ds(off, BATCH)],
                                      didx[h], isem[h])
                return c1, c2

            def idx_wait(j, h):
                pltpu.make_async_copy(src_h.at[pl.ds(base, BATCH)],
                                      sidx[h], isem[h]).wait()
                pltpu.make_async_copy(dst_h.at[pl.ds(base, BATCH)],
                                      didx[h], isem[h]).wait()

            def ge_start(j, h):
                pltpu.async_copy(tbl.at[sidx[h]], gbuf[h], gsem[h])
                off = base + j * BATCH
                pltpu.async_copy(etb.at[pl.ds(off, BATCH)], ebuf[h], esem[h])

            def ge_wait(j, h):
                pltpu.make_async_copy(tbl.at[sidx[h]], gbuf[h],
                                      gsem[h]).wait()
                pltpu.make_async_copy(etb.at[pl.ds(base, BATCH)], ebuf[h],
                                      esem[h]).wait()

            def process(j, h):
                ge_wait(j, h)
                off = j * BATCH
                for q in range(BATCH // 16):
                    dv = didx[h][pl.ds(q * 16, 16)] - nbase
                    pos = lax.iota(jnp.int32, 16) + (off + q * 16)
                    ok = (dv >= 0) & (dv < NPC) & (pos < EPT)
                    didx2[pl.ds(q * 16, 16)] = jnp.where(ok, dv, DUMMY)

                @pl.when(j + 2 < NBT)
                def _():
                    idx_start(j + 2, h)

                @pl.when(j + 1 < NBT)
                def _():
                    idx_wait(j + 1, 1 - h)
                    ge_start(j + 1, 1 - h)

                def edge_body(i, cc):
                    for v in range(CCH // 16):
                        g = gbuf[h][i, pl.ds(loff + v * 16, 16)]
                        e = ebuf[h][i, pl.ds(loff + v * 16, 16)]
                        m = jnp.maximum(g + e, 0.0) + 1e-7
                        p = jnp.exp(m * tvec)
                        stage[i, pl.ds(v * 16, 16)] = p * m
                        stage[i, pl.ds(CCH + v * 16, 16)] = p
                    return cc

                lax.fori_loop(0, BATCH, edge_body, 0)
                pltpu.sync_copy(stage, acc.at[didx2], add=True)

            idx_start(0, 0)
            idx_start(1, 1)
            idx_wait(0, 0)
            ge_start(0, 0)

            def pair_body(jj, carry):
                process(2 * jj, 0)
                process(2 * jj + 1, 1)
                return carry

            lax.fori_loop(0, NBT // 2, pair_body, 0)
            process(NBT - 1, (NBT - 1) % 2)
            plsc.subcore_barrier()
            pltpu.sync_copy(acc.at[pl.ds(rbase, RPT)],
                            po.at[c, k, pl.ds(rbase, RPT)])
            plsc.subcore_barrier()

    return sck(xs[0], xs[1], es[0], es[1], src, dst, t16, zeros)


def _tc2_body(po_ref, xd_ref, w1_ref, bg_ref, bb_ref, w2_ref,
              h_ref, s_ref, acc):
    i = pl.program_id(0)
    po = po_ref[...]
    n0 = jnp.concatenate([po[0, k, :, :CCH] for k in range(NCHUNK)], axis=-1)
    d0 = jnp.concatenate([po[0, k, :, CCH:] for k in range(NCHUNK)], axis=-1)
    n1 = jnp.concatenate([po[1, k, :, :CCH] for k in range(NCHUNK)], axis=-1)
    d1 = jnp.concatenate([po[1, k, :, CCH:] for k in range(NCHUNK)], axis=-1)
    c1 = i >= (NPC // 80)
    n = jnp.where(c1, n1, n0)
    d = jnp.where(c1, d1, d0) + 1e-16
    outb = n / d + xd_ref[...]
    h1 = jnp.dot(outb, w1_ref[...], preferred_element_type=_F32)
    bn_scale = bg_ref[...] * np.float32(1.0 / np.sqrt(1.0 + 1e-5))
    h1 = jnp.maximum(h1 * bn_scale + bb_ref[...], 0.0)
    h2 = jnp.dot(h1, w2_ref[...], preferred_element_type=_F32)
    h2 = jnp.maximum(h2, 0.0)
    h_ref[...] = h2
    s1 = jnp.sum(h2)
    s2 = jnp.sum(h2 * h2)

    @pl.when(i == 0)
    def _():
        acc[0] = s1
        acc[1] = s2

    @pl.when(i != 0)
    def _():
        acc[0] += s1
        acc[1] += s2

    @pl.when(i == pl.num_programs(0) - 1)
    def _():
        lane = lax.broadcasted_iota(jnp.int32, (1, 128), 1)
        s_ref[...] = jnp.where(lane == 0, acc[0],
                               jnp.where(lane == 1, acc[1], 0.0))


def _tc2(po, x_dst, w1, bn_gamma, bn_beta, w2):
    blk = 80
    grid = N // blk
    nblk = NPC // blk

    def po_map(i):
        return (0, 0, jnp.where(i < nblk, i, i - nblk), 0)

    return pl.pallas_call(
        _tc2_body,
        grid=(grid,),
        in_specs=[
            pl.BlockSpec((NC, NCHUNK, blk, 128), po_map),
            pl.BlockSpec((blk, D_HID), lambda i: (i, 0)),
            pl.BlockSpec((D_HID, 2 * D_HID), lambda i: (0, 0)),
            pl.BlockSpec((1, 2 * D_HID), lambda i: (0, 0)),
            pl.BlockSpec((1, 2 * D_HID), lambda i: (0, 0)),
            pl.BlockSpec((2 * D_HID, D_HID), lambda i: (0, 0)),
        ],
        out_specs=[
            pl.BlockSpec((blk, D_HID), lambda i: (i, 0)),
            pl.BlockSpec((1, 128), lambda i: (0, 0)),
        ],
        out_shape=[_sds((N, D_HID)), _sds((1, 128))],
        scratch_shapes=[pltpu.SMEM((2,), _F32)],
    )(po, x_dst, w1, bn_gamma.reshape(1, -1), bn_beta.reshape(1, -1), w2)


def _tc3_body(s_ref, h_ref, g_ref, b_ref, o_ref):
    cnt = np.float32(N * D_HID)
    mean = s_ref[0, 0] / cnt
    var = s_ref[0, 1] / cnt - mean * mean
    std = jnp.sqrt(jnp.maximum(var, 0.0))
    inv = 1.0 / (std + 1e-5)
    o_ref[...] = (h_ref[...] - mean) * inv * g_ref[...] + b_ref[...]


def _tc3(s, h, ln_gamma, ln_beta):
    blk = 1000
    grid = N // blk
    return pl.pallas_call(
        _tc3_body,
        grid=(grid,),
        in_specs=[
            pl.BlockSpec(memory_space=pltpu.SMEM),
            pl.BlockSpec((blk, D_HID), lambda i: (i, 0)),
            pl.BlockSpec((1, D_HID), lambda i: (0, 0)),
            pl.BlockSpec((1, D_HID), lambda i: (0, 0)),
        ],
        out_specs=pl.BlockSpec((blk, D_HID), lambda i: (i, 0)),
        out_shape=_sds((N, D_HID)),
    )(s, h, ln_gamma.reshape(1, -1), ln_beta.reshape(1, -1))


def kernel(x, edge_index, edge_attr, W_src, W_dst, W_edge, t,
           W_mlp1, bn_gamma, bn_beta, W_mlp2, ln_gamma, ln_beta):
    src = jnp.pad(edge_index[0], (0, 128))
    dst = jnp.pad(edge_index[1], (0, 128), constant_values=N)
    t16 = jnp.full((16,), t, _F32)
    xsa, xsb, x_dst = _tc0(x, W_src, W_dst)
    ea, eb = _tc1(edge_attr, W_edge)
    po = _sc_edge((xsa, xsb), (ea, eb), src, dst, t16)
    h_pre, s = _tc2(po, x_dst, W_mlp1, bn_gamma, bn_beta, W_mlp2)
    return _tc3(s, h_pre, ln_gamma, ln_beta)
